# baseline (speedup 1.0000x reference)
import functools

import jax
import jax.numpy as jnp
from jax import lax
from jax.experimental import pallas as pl
from jax.experimental.pallas import tpu as pltpu
from jax.experimental.pallas import tpu_sc as plsc

T = 8192
D = 4096
K = 8
E = 64
BT = 512
_NC = 2
_NS = 16
_NW = _NC * _NS
_TPW = T // _NW
_L = 16


def _eff_body(idx_hbm, gate_hbm, scales_hbm, out_hbm,
              scales_v, idx_v, gate_v, eff_v):
    wid = lax.axis_index("s") * _NC + lax.axis_index("c")
    base = wid * _TPW
    pltpu.sync_copy(scales_hbm, scales_v)
    pltpu.sync_copy(idx_hbm.at[:, pl.ds(base, _TPW)], idx_v)
    pltpu.sync_copy(gate_hbm.at[:, pl.ds(base, _TPW)], gate_v)
    for c in range(_TPW // _L):
        acc = jnp.zeros((_L,), jnp.float32)
        for k in range(K):
            i16 = idx_v[k, pl.ds(c * _L, _L)]
            g16 = gate_v[k, pl.ds(c * _L, _L)]
            s16 = plsc.load_gather(scales_v, [i16])
            acc = acc + g16 * s16
        eff_v[pl.ds(c * _L, _L)] = acc
    pltpu.sync_copy(eff_v, out_hbm.at[pl.ds(base, _TPW)])


_eff_kernel = functools.partial(
    pl.kernel,
    mesh=plsc.VectorSubcoreMesh(core_axis_name="c", subcore_axis_name="s"),
    out_type=jax.ShapeDtypeStruct((T,), jnp.float32),
    scratch_types=[
        pltpu.VMEM((E,), jnp.float32),
        pltpu.VMEM((K, _TPW), jnp.int32),
        pltpu.VMEM((K, _TPW), jnp.float32),
        pltpu.VMEM((_TPW,), jnp.float32),
    ],
    compiler_params=pltpu.CompilerParams(needs_layout_passes=False),
)(_eff_body)


def _scale_body(eff_ref, h_ref, out_ref):
    out_ref[...] = eff_ref[...] * h_ref[...]


@jax.jit
def kernel(h, top_k_experts, expert_gate, scales):
    idx_t = top_k_experts.astype(jnp.int32).T
    gate_t = expert_gate.T
    eff = _eff_kernel(idx_t, gate_t, scales)
    eff2 = eff.reshape(T, 1)
    return pl.pallas_call(
        _scale_body,
        grid=(T // BT,),
        in_specs=[
            pl.BlockSpec((BT, 1), lambda i: (i, 0)),
            pl.BlockSpec((BT, D), lambda i: (i, 0)),
        ],
        out_specs=pl.BlockSpec((BT, D), lambda i: (i, 0)),
        out_shape=jax.ShapeDtypeStruct((T, D), jnp.float32),
    )(eff2, h)
